# Initial kernel scaffold; baseline (speedup 1.0000x reference)
import jax
import jax.numpy as jnp
from jax.experimental import pallas as pl


def kernel(features, edge_index, dense_graph, W_emb, b_emb, conv_W, conv_b, lin_W, W_ro, b_ro, W_adj, b_adj, W_as, b_as):
    raise NotImplementedError("write your pallas kernel here")


# speedup vs baseline: 3.9368x; 3.9368x over previous
import functools

import jax
import jax.numpy as jnp
from jax import lax
from jax.experimental import pallas as pl
from jax.experimental.pallas import tpu as pltpu
from jax.experimental.pallas import tpu_sc as plsc

N = 10000
D = 128
E = 320000
GAMMA = 0.1
EPS = 0.1
NUM_ITERS = 4

BI = 2000
NBI = N // BI

_DN = (((1,), (1,)), ((), ()))


def _mm(x, w):
    return lax.dot_general(x, w, _DN, preferred_element_type=jnp.float32)


def _prologue_body(feat, w_emb, b_emb, lin_w, a_mat, conv_b, x_o, ne_o, xa_o):
    x = _mm(feat[...], w_emb[...]) + b_emb[...]
    x_o[...] = x
    ne_o[...] = _mm(x, lin_w[...])
    xa_o[...] = _mm(x, a_mat[...]) + conv_b[...]


def _update_body(x, xa, p0, p1, lin_w, a_mat, conv_b, x_o, ne_o, xa_o):
    xn = x[...] + EPS * jnp.tanh(xa[...] + p0[...] + p1[...])
    x_o[...] = xn
    ne_o[...] = _mm(xn, lin_w[...])
    xa_o[...] = _mm(xn, a_mat[...]) + conv_b[...]


def _final_update_body(x, xa, p0, p1, x_o):
    x_o[...] = x[...] + EPS * jnp.tanh(xa[...] + p0[...] + p1[...])


_row = pl.BlockSpec((BI, D), lambda i: (i, 0))
_wfull = pl.BlockSpec((D, D), lambda i: (0, 0))
_brow = pl.BlockSpec((1, D), lambda i: (0, 0))
_nd_f32 = jax.ShapeDtypeStruct((N, D), jnp.float32)


def _prologue_call(features, W_emb, b_emb2, lin_W, A, conv_b2):
    return pl.pallas_call(
        _prologue_body,
        grid=(NBI,),
        in_specs=[_row, _wfull, _brow, _wfull, _wfull, _brow],
        out_specs=[_row, _row, _row],
        out_shape=[_nd_f32, _nd_f32, _nd_f32],
    )(features, W_emb, b_emb2, lin_W, A, conv_b2)


def _update_call(x, xa, p0, p1, lin_W, A, conv_b2):
    return pl.pallas_call(
        _update_body,
        grid=(NBI,),
        in_specs=[_row, _row, _row, _row, _wfull, _wfull, _brow],
        out_specs=[_row, _row, _row],
        out_shape=[_nd_f32, _nd_f32, _nd_f32],
    )(x, xa, p0, p1, lin_W, A, conv_b2)


def _final_update_call(x, xa, p0, p1):
    return pl.pallas_call(
        _final_update_body,
        grid=(NBI,),
        in_specs=[_row, _row, _row, _row],
        out_specs=_row,
        out_shape=_nd_f32,
    )(x, xa, p0, p1)


BL_I = 1000
BL_J = 1024
NLI = N // BL_I
NLJ = (N + BL_J - 1) // BL_J


def _loss_body(x, feat, g, w_adj, b_adj, w_ro, b_ro, out, acc):
    i = pl.program_id(0)
    j = pl.program_id(1)

    @pl.when((i == 0) & (j == 0))
    def _init():
        acc[0] = 0.0
        acc[1] = 0.0

    p = _mm(x[...], w_adj[...]) + b_adj[...]
    dd = jnp.maximum(p, 0.0) - g[...]
    col = j * BL_J + lax.broadcasted_iota(jnp.int32, (BL_I, BL_J), 1)
    dd = jnp.where(col < N, dd, 0.0)
    acc[0] += jnp.sum(dd * dd)

    @pl.when(j == 0)
    def _feat_loss():
        y = jnp.maximum(_mm(x[...], w_ro[...]) + b_ro[...], 0.0) - feat[...]
        acc[1] += jnp.sum(y * y)

    @pl.when((i == NLI - 1) & (j == NLJ - 1))
    def _fin():
        out[...] = jnp.full((1, 1), acc[0] / (N * N) + acc[1] / (N * D),
                            jnp.float32)


def _loss_call(x, features, dense_graph, W_adj, b_adj2, W_ro, b_ro2):
    return pl.pallas_call(
        _loss_body,
        grid=(NLI, NLJ),
        in_specs=[
            pl.BlockSpec((BL_I, D), lambda i, j: (i, 0)),
            pl.BlockSpec((BL_I, D), lambda i, j: (i, 0)),
            pl.BlockSpec((BL_I, BL_J), lambda i, j: (i, j)),
            pl.BlockSpec((BL_J, D), lambda i, j: (j, 0)),
            pl.BlockSpec((1, BL_J), lambda i, j: (0, j)),
            pl.BlockSpec((D, D), lambda i, j: (0, 0)),
            pl.BlockSpec((1, D), lambda i, j: (0, 0)),
        ],
        out_specs=pl.BlockSpec((1, 1), lambda i, j: (0, 0)),
        out_shape=jax.ShapeDtypeStruct((1, 1), jnp.float32),
        scratch_shapes=[pltpu.SMEM((2,), jnp.float32)],
    )(x, features, dense_graph, W_adj, b_adj2, W_ro, b_ro2)


NTILES = 16
NCORES = 2
CHUNK = 80
EPW = E // (NTILES * NCORES)
NCH = EPW // CHUNK
NPAD = 10240
RPT = NPAD // NTILES


def _seg_sum(vals, src, dst, zeros):
    mesh = plsc.VectorSubcoreMesh(core_axis_name="c", subcore_axis_name="s")

    @functools.partial(
        pl.kernel,
        out_type=[jax.ShapeDtypeStruct((NPAD, D), jnp.float32)] * NCORES,
        mesh=mesh,
        scratch_types=[
            pltpu.VMEM((CHUNK,), jnp.int32),
            pltpu.VMEM((CHUNK,), jnp.int32),
            pltpu.VMEM((CHUNK, D), jnp.float32),
            pltpu.VMEM_SHARED((NPAD, D), jnp.float32),
            pltpu.SemaphoreType.DMA,
        ],
    )
    def seg_kernel(vals_hbm, src_hbm, dst_hbm, zeros_hbm, out0_hbm, out1_hbm,
                   src_v, dst_v, rows_v, acc, sem):
        c = lax.axis_index("c")
        s = lax.axis_index("s")
        r0 = pl.multiple_of(s * RPT, 8)
        pltpu.sync_copy(zeros_hbm.at[pl.ds(r0, RPT)], acc.at[pl.ds(r0, RPT)])
        plsc.subcore_barrier()
        base = (c * NTILES + s) * EPW

        def body(i, carry):
            off = pl.multiple_of(base + i * CHUNK, 8)
            pltpu.sync_copy(src_hbm.at[pl.ds(off, CHUNK)], src_v)
            pltpu.sync_copy(dst_hbm.at[pl.ds(off, CHUNK)], dst_v)
            pltpu.async_copy(vals_hbm.at[src_v], rows_v, sem).wait()
            pltpu.sync_copy(rows_v, acc.at[dst_v], add=True)
            return carry

        lax.fori_loop(0, NCH, body, 0)
        plsc.subcore_barrier()

        @pl.when(c == 0)
        def _w0():
            pltpu.sync_copy(acc.at[pl.ds(r0, RPT)], out0_hbm.at[pl.ds(r0, RPT)])

        @pl.when(c == 1)
        def _w1():
            pltpu.sync_copy(acc.at[pl.ds(r0, RPT)], out1_hbm.at[pl.ds(r0, RPT)])

    return seg_kernel(vals, src, dst, zeros)


def kernel(features, edge_index, dense_graph, W_emb, b_emb, conv_W, conv_b,
           lin_W, W_ro, b_ro, W_adj, b_adj, W_as, b_as):
    A = conv_W - conv_W.T - GAMMA * jnp.eye(D, dtype=jnp.float32)
    b_emb2 = b_emb.reshape(1, D)
    conv_b2 = conv_b.reshape(1, D)
    b_ro2 = b_ro.reshape(1, D)
    b_adj2 = b_adj.reshape(1, N)
    src = edge_index[0]
    dst = edge_index[1]
    zeros = jnp.zeros((NPAD, D), jnp.float32)

    x, ne, xa = _prologue_call(features, W_emb, b_emb2, lin_W, A, conv_b2)
    for it in range(NUM_ITERS):
        p0, p1 = _seg_sum(ne, src, dst, zeros)
        if it < NUM_ITERS - 1:
            x, ne, xa = _update_call(x, xa, p0, p1, lin_W, A, conv_b2)
        else:
            x = _final_update_call(x, xa, p0, p1)
    out = _loss_call(x, features, dense_graph, W_adj, b_adj2, W_ro, b_ro2)
    return out[0, 0]


# speedup vs baseline: 8.2639x; 2.0991x over previous
import functools

import jax
import jax.numpy as jnp
from jax import lax
from jax.experimental import pallas as pl
from jax.experimental.pallas import tpu as pltpu
from jax.experimental.pallas import tpu_sc as plsc

N = 10000
D = 128
E = 320000
GAMMA = 0.1
EPS = 0.1
NUM_ITERS = 4

BI = 2000
NBI = N // BI

_DN = (((1,), (1,)), ((), ()))


def _mm(x, w):
    return lax.dot_general(x, w, _DN, preferred_element_type=jnp.float32)


def _prologue_body(feat, w_emb, b_emb, lin_w, a_mat, conv_b, x_o, ne_o, xa_o):
    x = _mm(feat[...], w_emb[...]) + b_emb[...]
    x_o[...] = x
    ne_o[...] = _mm(x, lin_w[...])
    xa_o[...] = _mm(x, a_mat[...]) + conv_b[...]


def _update_body(x, xa, p0, p1, lin_w, a_mat, conv_b, x_o, ne_o, xa_o):
    xn = x[...] + EPS * jnp.tanh(xa[...] + p0[...] + p1[...])
    x_o[...] = xn
    ne_o[...] = _mm(xn, lin_w[...])
    xa_o[...] = _mm(xn, a_mat[...]) + conv_b[...]


def _final_update_body(x, xa, p0, p1, x_o):
    x_o[...] = x[...] + EPS * jnp.tanh(xa[...] + p0[...] + p1[...])


_row = pl.BlockSpec((BI, D), lambda i: (i, 0))
_wfull = pl.BlockSpec((D, D), lambda i: (0, 0))
_brow = pl.BlockSpec((1, D), lambda i: (0, 0))
_nd_f32 = jax.ShapeDtypeStruct((N, D), jnp.float32)


def _prologue_call(features, W_emb, b_emb2, lin_W, A, conv_b2):
    return pl.pallas_call(
        _prologue_body,
        grid=(NBI,),
        in_specs=[_row, _wfull, _brow, _wfull, _wfull, _brow],
        out_specs=[_row, _row, _row],
        out_shape=[_nd_f32, _nd_f32, _nd_f32],
    )(features, W_emb, b_emb2, lin_W, A, conv_b2)


def _update_call(x, xa, p0, p1, lin_W, A, conv_b2):
    return pl.pallas_call(
        _update_body,
        grid=(NBI,),
        in_specs=[_row, _row, _row, _row, _wfull, _wfull, _brow],
        out_specs=[_row, _row, _row],
        out_shape=[_nd_f32, _nd_f32, _nd_f32],
    )(x, xa, p0, p1, lin_W, A, conv_b2)


def _final_update_call(x, xa, p0, p1):
    return pl.pallas_call(
        _final_update_body,
        grid=(NBI,),
        in_specs=[_row, _row, _row, _row],
        out_specs=_row,
        out_shape=_nd_f32,
    )(x, xa, p0, p1)


BL_I = 1000
BL_J = 1024
NLI = N // BL_I
NLJ = (N + BL_J - 1) // BL_J


def _loss_body(x, feat, g, w_adj, b_adj, w_ro, b_ro, out, acc):
    i = pl.program_id(0)
    j = pl.program_id(1)

    @pl.when((i == 0) & (j == 0))
    def _init():
        acc[0] = 0.0
        acc[1] = 0.0

    p = _mm(x[...], w_adj[...]) + b_adj[...]
    dd = jnp.maximum(p, 0.0) - g[...]
    col = j * BL_J + lax.broadcasted_iota(jnp.int32, (BL_I, BL_J), 1)
    dd = jnp.where(col < N, dd, 0.0)
    acc[0] += jnp.sum(dd * dd)

    @pl.when(j == 0)
    def _feat_loss():
        y = jnp.maximum(_mm(x[...], w_ro[...]) + b_ro[...], 0.0) - feat[...]
        acc[1] += jnp.sum(y * y)

    @pl.when((i == NLI - 1) & (j == NLJ - 1))
    def _fin():
        out[...] = jnp.full((1, 1), acc[0] / (N * N) + acc[1] / (N * D),
                            jnp.float32)


def _loss_call(x, features, dense_graph, W_adj, b_adj2, W_ro, b_ro2):
    return pl.pallas_call(
        _loss_body,
        grid=(NLI, NLJ),
        in_specs=[
            pl.BlockSpec((BL_I, D), lambda i, j: (i, 0)),
            pl.BlockSpec((BL_I, D), lambda i, j: (i, 0)),
            pl.BlockSpec((BL_I, BL_J), lambda i, j: (i, j)),
            pl.BlockSpec((BL_J, D), lambda i, j: (j, 0)),
            pl.BlockSpec((1, BL_J), lambda i, j: (0, j)),
            pl.BlockSpec((D, D), lambda i, j: (0, 0)),
            pl.BlockSpec((1, D), lambda i, j: (0, 0)),
        ],
        out_specs=pl.BlockSpec((1, 1), lambda i, j: (0, 0)),
        out_shape=jax.ShapeDtypeStruct((1, 1), jnp.float32),
        scratch_shapes=[pltpu.SMEM((2,), jnp.float32)],
    )(x, features, dense_graph, W_adj, b_adj2, W_ro, b_ro2)


NTILES = 16
NCORES = 2
CHUNK = 128
EPW = E // (NTILES * NCORES)
NCH = EPW // CHUNK
TAIL = EPW - NCH * CHUNK
DEPTH = 2
NPAD = 10240
RPT = NPAD // NTILES


def _seg_sum(vals, src, dst, zeros):
    mesh = plsc.VectorSubcoreMesh(core_axis_name="c", subcore_axis_name="s")

    @functools.partial(
        pl.kernel,
        out_type=[jax.ShapeDtypeStruct((NPAD, D), jnp.float32)] * NCORES,
        mesh=mesh,
        scratch_types=[
            pltpu.VMEM((EPW,), jnp.int32),
            [pltpu.VMEM((CHUNK,), jnp.int32)] * DEPTH,
            [pltpu.VMEM((CHUNK, D), jnp.float32)] * DEPTH,
            pltpu.VMEM((TAIL,), jnp.int32),
            pltpu.VMEM((TAIL, D), jnp.float32),
            pltpu.VMEM_SHARED((NPAD, D), jnp.float32),
            [pltpu.SemaphoreType.DMA] * DEPTH,
            [pltpu.SemaphoreType.DMA] * DEPTH,
        ],
    )
    def seg_kernel(vals_hbm, src_hbm, dst_hbm, zeros_hbm, out0_hbm, out1_hbm,
                   src_t, dst_v, rows_v, dst_tl, rows_tl, acc, gsem, isem):
        c = lax.axis_index("c")
        s = lax.axis_index("s")
        r0 = pl.multiple_of(s * RPT, 8)
        base = pl.multiple_of((c * NTILES + s) * EPW, 8)
        pltpu.sync_copy(zeros_hbm.at[pl.ds(r0, RPT)], acc.at[pl.ds(r0, RPT)])
        pltpu.sync_copy(src_hbm.at[pl.ds(base, EPW)], src_t)
        plsc.subcore_barrier()

        def fetch(i, b):
            hoff = pl.multiple_of(base + i * CHUNK, 8)
            loff = pl.multiple_of(i * CHUNK, 8)
            pltpu.async_copy(dst_hbm.at[pl.ds(hoff, CHUNK)], dst_v[b], isem[b])
            pltpu.async_copy(vals_hbm.at[src_t.at[pl.ds(loff, CHUNK)]],
                             rows_v[b], gsem[b])

        for b in range(DEPTH):
            fetch(b, b)

        def visit(i, b, prefetch):
            hoff = pl.multiple_of(base + i * CHUNK, 8)
            loff = pl.multiple_of(i * CHUNK, 8)
            pltpu.make_async_copy(dst_hbm.at[pl.ds(hoff, CHUNK)],
                                  dst_v[b], isem[b]).wait()
            pltpu.make_async_copy(vals_hbm.at[src_t.at[pl.ds(loff, CHUNK)]],
                                  rows_v[b], gsem[b]).wait()
            pltpu.sync_copy(rows_v[b], acc.at[dst_v[b]], add=True)
            if prefetch:
                fetch(i + DEPTH, b)

        @pl.loop(0, NCH - 2 * DEPTH + 1, step=DEPTH)
        def _grp(g):
            for b in range(DEPTH):
                visit(g + b, b, prefetch=True)

        for i in range(NCH - DEPTH, NCH):
            visit(i, i % DEPTH, prefetch=False)

        toff = pl.multiple_of(base + NCH * CHUNK, 8)
        tloff = pl.multiple_of(NCH * CHUNK, 8)
        pltpu.sync_copy(dst_hbm.at[pl.ds(toff, TAIL)], dst_tl)
        pltpu.async_copy(vals_hbm.at[src_t.at[pl.ds(tloff, TAIL)]],
                         rows_tl, gsem[0]).wait()
        pltpu.sync_copy(rows_tl, acc.at[dst_tl], add=True)

        plsc.subcore_barrier()

        @pl.when(c == 0)
        def _w0():
            pltpu.sync_copy(acc.at[pl.ds(r0, RPT)], out0_hbm.at[pl.ds(r0, RPT)])

        @pl.when(c == 1)
        def _w1():
            pltpu.sync_copy(acc.at[pl.ds(r0, RPT)], out1_hbm.at[pl.ds(r0, RPT)])

    return seg_kernel(vals, src, dst, zeros)


def kernel(features, edge_index, dense_graph, W_emb, b_emb, conv_W, conv_b,
           lin_W, W_ro, b_ro, W_adj, b_adj, W_as, b_as):
    A = conv_W - conv_W.T - GAMMA * jnp.eye(D, dtype=jnp.float32)
    b_emb2 = b_emb.reshape(1, D)
    conv_b2 = conv_b.reshape(1, D)
    b_ro2 = b_ro.reshape(1, D)
    b_adj2 = b_adj.reshape(1, N)
    src = edge_index[0]
    dst = edge_index[1]
    zeros = jnp.zeros((NPAD, D), jnp.float32)

    x, ne, xa = _prologue_call(features, W_emb, b_emb2, lin_W, A, conv_b2)
    for it in range(NUM_ITERS):
        p0, p1 = _seg_sum(ne, src, dst, zeros)
        if it < NUM_ITERS - 1:
            x, ne, xa = _update_call(x, xa, p0, p1, lin_W, A, conv_b2)
        else:
            x = _final_update_call(x, xa, p0, p1)
    out = _loss_call(x, features, dense_graph, W_adj, b_adj2, W_ro, b_ro2)
    return out[0, 0]
